# speedup vs baseline: 1.2967x; 1.2967x over previous
import functools

import jax
import jax.numpy as jnp
from jax.experimental import pallas as pl
from jax.experimental.pallas import tpu as pltpu

IN_DIM = 128
HID = 128
OUT = 64
KC = 64
KN = 10
NCLS = 2


def _lrelu(x):
    return jnp.where(x >= 0, x, 0.01 * x)


_PNAMES = [
    'enc_W', 'enc_b', 'cl_W', 'cl_b', 'head_W', 'head_b', 'tail_W', 'tail_b',
    'sage_l_W', 'sage_l_b', 'sage_r_W', 'lin_sum_W', 'lin_sum_b',
    'lin_bi_W', 'lin_bi_b', 'gU_W', 'gU_b', 'gV_W', 'gV_b', 'gW_W', 'gW_b',
    'ln_g', 'ln_b', 'feat_W', 'feat_b', 'attnA_W', 'attnB_W', 'attnC_W',
    'gate1_W', 'gate1_b', 'gate2_W', 'gate2_b',
    'fus_gate_Wa', 'fus_gate_Wb', 'fus_gate_b',
    'fus_tr_Wa', 'fus_tr_Wb', 'fus_tr_b',
    'bl_W', 'bl_b', 'c1_W', 'c1_b', 'c2_W', 'c2_b',
]


def _body(nblk, x_ref, gu_ref, *rest):
    p = {name: rest[k] for k, name in enumerate(_PNAMES)}
    out_ref = rest[len(_PNAMES)]
    parts_ref, num_ref, md_ref = rest[len(_PNAMES) + 1:]
    i = pl.program_id(0)

    @pl.when(i == 0)
    def _init():
        parts_ref[...] = jnp.zeros_like(parts_ref)
        num_ref[...] = jnp.zeros_like(num_ref)
        md_ref[0] = -1e30
        md_ref[1] = 0.0

    x = x_ref[...]
    H = _lrelu(x @ p['enc_W'][...] + p['enc_b'][...])

    cl = H @ p['cl_W'][...] + p['cl_b'][...]
    g = -jnp.log(-jnp.log(gu_ref[...]))
    z = (cl + g) * 2.0
    z = z - jnp.max(z, axis=-1, keepdims=True)
    ez = jnp.exp(z)
    P = ez / jnp.sum(ez, axis=-1, keepdims=True)
    parts_ref[...] += P.T @ H

    hf = _lrelu(H @ p['feat_W'][...] + p['feat_b'][...])
    a = _lrelu(hf @ p['attnA_W'][...])
    b = jax.nn.sigmoid(hf @ p['attnB_W'][...])
    s = (a * b) @ p['attnC_W'][...]
    m_old = md_ref[0]
    d_old = md_ref[1]
    m_new = jnp.maximum(m_old, jnp.max(s))
    e = jnp.exp(s - m_new)
    scale = jnp.exp(m_old - m_new)
    md_ref[0] = m_new
    md_ref[1] = d_old * scale + jnp.sum(e)
    num_ref[...] = num_ref[...] * scale + e.T @ hf

    @pl.when(i == nblk - 1)
    def _final():
        parts = parts_ref[...]
        attn_feat = num_ref[...] / md_ref[1]

        e_h = parts @ p['head_W'][...] + p['head_b'][...]
        e_t = parts @ p['tail_W'][...] + p['tail_b'][...]
        sim = (e_h @ e_t.T) * (HID ** -0.5)

        col = jax.lax.broadcasted_iota(jnp.int32, (KC, KC), 1)
        masked = sim
        E = jnp.zeros((KC, KC), jnp.float32)
        U = jnp.zeros((KC, KC), jnp.float32)
        wden = jnp.zeros((KC, 1), jnp.float32)
        v0 = None
        for k in range(KN):
            v = jnp.max(masked, axis=1, keepdims=True)
            cand = masked == v
            idx = jnp.min(jnp.where(cand, col, KC), axis=1, keepdims=True)
            onehot = (col == idx).astype(jnp.float32)
            if k == 0:
                v0 = v
            ev = jnp.exp(v - v0)
            E = E + onehot
            U = U + ev * onehot
            wden = wden + ev
            masked = jnp.where(onehot > 0, -1e30, masked)
        W_edge = U / wden

        cnt = jnp.sum(E.T, axis=1, keepdims=True)
        agg = (E.T @ parts) / jnp.maximum(cnt, 1.0)
        Xg = _lrelu(agg @ p['sage_l_W'][...] + p['sage_l_b'][...]
                    + parts @ p['sage_r_W'][...])

        summed = W_edge @ Xg
        sum_msg = (Xg + summed) @ p['lin_sum_W'][...] + p['lin_sum_b'][...]
        bi_msg = (Xg * summed) @ p['lin_bi_W'][...] + p['lin_bi_b'][...]
        u = Xg @ p['gU_W'][...] + p['gU_b'][...]
        vv = summed @ p['gV_W'][...] + p['gV_b'][...]
        gate = jax.nn.sigmoid((u + vv) @ p['gW_W'][...] + p['gW_b'][...])
        outg = _lrelu(gate * sum_msg + (1 - gate) * bi_msg)
        res = outg + Xg

        mu = jnp.mean(res, axis=1, keepdims=True)
        var = jnp.mean((res - mu) ** 2, axis=1, keepdims=True)
        gnn = (res - mu) / jnp.sqrt(var + 1e-5) * p['ln_g'][...] + p['ln_b'][...]

        gate_sc = (_lrelu(gnn @ p['gate1_W'][...] + p['gate1_b'][...])
                   @ p['gate2_W'][...] + p['gate2_b'][...])
        ga = jnp.exp(gate_sc - jnp.max(gate_sc))
        alpha = ga / jnp.sum(ga)
        graph_feat = alpha.T @ gnn

        af = _lrelu(attn_feat @ p['bl_W'][...] + p['bl_b'][...])

        fg = jax.nn.sigmoid(graph_feat @ p['fus_gate_Wa'][...]
                            + af @ p['fus_gate_Wb'][...]
                            + p['fus_gate_b'][...])
        tr = _lrelu(graph_feat @ p['fus_tr_Wa'][...]
                    + af @ p['fus_tr_Wb'][...]
                    + p['fus_tr_b'][...])
        fused = fg * graph_feat + (1 - fg) * tr

        logits = (_lrelu(fused @ p['c1_W'][...] + p['c1_b'][...])
                  @ p['c2_W'][...] + p['c2_b'][...])
        out_ref[...] = logits


def kernel(X, params, gumbel_u):
    N = X.shape[0]
    B = None
    for cand in (5000, 2500, 2000, 1000, 500, 250, 200, 100, 50, 25, 8, 1):
        if N % cand == 0:
            B = cand
            break
    nblk = N // B

    q = dict(params)
    for name in list(q):
        if q[name].ndim == 1:
            q[name] = q[name].reshape(1, -1)
    q['fus_gate_Wa'] = params['fus_gate_W'][:OUT]
    q['fus_gate_Wb'] = params['fus_gate_W'][OUT:]
    q['fus_tr_Wa'] = params['fus_tr_W'][:OUT]
    q['fus_tr_Wb'] = params['fus_tr_W'][OUT:]
    plist = [q[name] for name in _PNAMES]

    def const_spec(arr):
        return pl.BlockSpec(arr.shape, lambda i: (0,) * arr.ndim)

    in_specs = [
        pl.BlockSpec((B, IN_DIM), lambda i: (i, 0)),
        pl.BlockSpec((B, KC), lambda i: (i, 0)),
    ] + [const_spec(arr) for arr in plist]

    out = pl.pallas_call(
        functools.partial(_body, nblk),
        grid=(nblk,),
        in_specs=in_specs,
        out_specs=pl.BlockSpec((1, NCLS), lambda i: (0, 0)),
        out_shape=jax.ShapeDtypeStruct((1, NCLS), jnp.float32),
        scratch_shapes=[
            pltpu.VMEM((KC, HID), jnp.float32),
            pltpu.VMEM((1, HID), jnp.float32),
            pltpu.SMEM((2,), jnp.float32),
        ],
    )(X, gumbel_u, *plist)
    return out
